# Initial kernel scaffold; baseline (speedup 1.0000x reference)
import jax
import jax.numpy as jnp
from jax.experimental import pallas as pl


def kernel(x, src0, dst0, src1, dst1, W0a, b0a, bn0_w, bn0_b, W0b, b0b, W1a, b1a, bn1_w, bn1_b, W1b, b1b, lin1_w, lin1_b, lin2_w, lin2_b):
    raise NotImplementedError("write your pallas kernel here")


# speedup vs baseline: 3.8641x; 3.8641x over previous
import functools

import jax
import jax.numpy as jnp
from jax import lax
from jax.experimental import pallas as pl
from jax.experimental.pallas import tpu as pltpu
from jax.experimental.pallas import tpu_sc as plsc

N0, N1, N2 = 100000, 20000, 4096
D = 128
E0, E1 = 320000, 65536
BN_EPS = 1e-5

NC, NS, L = 2, 16, 16
CH = 128

HALF0 = N1 // NC
DUMMY0 = HALF0
ACC0 = 10240
ROWS0 = E0 // CH
ITER0 = (ROWS0 + NS - 1) // NS

ACC1 = N2
ROWS1 = E1 // CH
ITER1 = ROWS1 // (NC * NS)


def _agg_mesh_kernel(body, out_rows, zero_rows):
    mesh = plsc.VectorSubcoreMesh(core_axis_name="c", subcore_axis_name="s")
    return pl.kernel(
        body,
        out_type=jax.ShapeDtypeStruct((out_rows, D), jnp.float32),
        mesh=mesh,
        scratch_types=[
            pltpu.VMEM((CH,), jnp.int32),
            pltpu.VMEM((CH,), jnp.int32),
            pltpu.VMEM((1, CH), jnp.int32),
            pltpu.VMEM((CH, D), jnp.float32),
            pltpu.VMEM_SHARED((zero_rows * NS, D), jnp.float32),
            pltpu.SemaphoreType.DMA,
        ],
    )


def _agg0_body(x_hbm, src_hbm, dst_hbm, zeros_hbm, out_hbm,
               sidx_v, draw_v, didx_v, rows_v, acc, sem):
    cid = lax.axis_index("c")
    sid = lax.axis_index("s")
    zrows = ACC0 // NS

    pltpu.sync_copy(zeros_hbm, acc.at[pl.ds(sid * zrows, zrows)])
    plsc.subcore_barrier()

    base = cid * HALF0

    def step(i, carry):
        r = sid + NS * i

        @pl.when(r < ROWS0)
        def _():
            pltpu.sync_copy(src_hbm.at[r], sidx_v)
            pltpu.sync_copy(dst_hbm.at[r], draw_v)
            pltpu.async_copy(x_hbm.at[sidx_v], rows_v, sem).wait()
            for k in range(CH // L):
                d = draw_v[pl.ds(k * L, L)]
                local = d - base
                oob = (local < 0) | (local >= HALF0)
                didx_v[0, pl.ds(k * L, L)] = jnp.where(oob, DUMMY0, local)
            pltpu.sync_copy(rows_v, acc.at[didx_v.at[0]], add=True)

        return carry

    lax.fori_loop(0, ITER0, step, 0)
    plsc.subcore_barrier()

    @pl.when(sid < NS - 1)
    def _():
        pltpu.sync_copy(acc.at[pl.ds(sid * 632, 632)],
                        out_hbm.at[pl.ds(cid * HALF0 + sid * 632, 632)])

    @pl.when(sid == NS - 1)
    def _():
        pltpu.sync_copy(acc.at[pl.ds((NS - 1) * 632, 520)],
                        out_hbm.at[pl.ds(cid * HALF0 + (NS - 1) * 632, 520)])


def _agg1_body(h_hbm, src_hbm, dst_hbm, zeros_hbm, out_hbm,
               sidx_v, draw_v, didx_v, rows_v, acc, sem):
    cid = lax.axis_index("c")
    sid = lax.axis_index("s")
    zrows = ACC1 // NS

    pltpu.sync_copy(zeros_hbm, acc.at[pl.ds(sid * zrows, zrows)])
    plsc.subcore_barrier()

    def step(i, carry):
        r = cid * (ROWS1 // NC) + sid + NS * i
        pltpu.sync_copy(src_hbm.at[r], sidx_v)
        pltpu.sync_copy(dst_hbm.at[r], draw_v)
        pltpu.async_copy(h_hbm.at[sidx_v], rows_v, sem).wait()
        for k in range(CH // L):
            didx_v[0, pl.ds(k * L, L)] = draw_v[pl.ds(k * L, L)]
        pltpu.sync_copy(rows_v, acc.at[didx_v.at[0]], add=True)
        return carry

    lax.fori_loop(0, ITER1, step, 0)
    plsc.subcore_barrier()

    frows = ACC1 // NS
    pltpu.sync_copy(acc.at[pl.ds(sid * frows, frows)],
                    out_hbm.at[pl.ds(cid * ACC1 + sid * frows, frows)])


def _mlp_body(a_ref, x_ref, w1_ref, b1_ref, w2_ref, b2_ref, o_ref):
    h = a_ref[...] + x_ref[...]
    h = jnp.dot(h, w1_ref[...], preferred_element_type=jnp.float32) + b1_ref[...]
    h = jnp.maximum(h, 0.0)
    h = jnp.dot(h, w2_ref[...], preferred_element_type=jnp.float32) + b2_ref[...]
    o_ref[...] = jnp.maximum(h, 0.0)


def _head_body(p0_ref, p1_ref, x_ref, w1_ref, b1_ref, w2_ref, b2_ref,
               l1_ref, c1_ref, l2_ref, c2_ref, o_ref):
    h = p0_ref[...] + p1_ref[...] + x_ref[...]
    h = jnp.dot(h, w1_ref[...], preferred_element_type=jnp.float32) + b1_ref[...]
    h = jnp.maximum(h, 0.0)
    h = jnp.dot(h, w2_ref[...], preferred_element_type=jnp.float32) + b2_ref[...]
    h = jnp.maximum(h, 0.0)
    h = jnp.dot(h, l1_ref[...], preferred_element_type=jnp.float32) + c1_ref[...]
    h = jnp.maximum(h, 0.0)
    z = jnp.dot(h, l2_ref[...], preferred_element_type=jnp.float32) + c2_ref[...]
    m = jnp.max(z, axis=-1, keepdims=True)
    e = jnp.exp(z - m)
    s = jnp.sum(e, axis=-1, keepdims=True)
    o_ref[...] = z - m - jnp.log(s)


def _full(shape):
    return pl.BlockSpec(shape, lambda i: (0, 0))


def _mlp(aggr, xt, w1, b1, w2, b2, rows, block):
    grid = (rows // block,)
    row_spec = pl.BlockSpec((block, D), lambda i: (i, 0))
    return pl.pallas_call(
        _mlp_body,
        grid=grid,
        in_specs=[row_spec, row_spec, _full((D, D)), _full((1, D)),
                  _full((D, D)), _full((1, D))],
        out_specs=row_spec,
        out_shape=jax.ShapeDtypeStruct((rows, D), jnp.float32),
    )(aggr, xt, w1, b1, w2, b2)


def _head(p0, p1, xt, w1, b1, w2, b2, l1, c1, l2, c2, rows, block, dout):
    grid = (rows // block,)
    row_spec = pl.BlockSpec((block, D), lambda i: (i, 0))
    out_spec = pl.BlockSpec((block, dout), lambda i: (i, 0))
    return pl.pallas_call(
        _head_body,
        grid=grid,
        in_specs=[row_spec, row_spec, row_spec,
                  _full((D, D)), _full((1, D)), _full((D, D)), _full((1, D)),
                  _full((D, D)), _full((1, D)), _full((D, dout)), _full((1, dout))],
        out_specs=out_spec,
        out_shape=jax.ShapeDtypeStruct((rows, dout), jnp.float32),
    )(p0, p1, xt, w1, b1, w2, b2, l1, c1, l2, c2)


@jax.jit
def kernel(x, src0, dst0, src1, dst1, W0a, b0a, bn0_w, bn0_b, W0b, b0b,
           W1a, b1a, bn1_w, bn1_b, W1b, b1b, lin1_w, lin1_b, lin2_w, lin2_b):
    x = x.astype(jnp.float32)
    src0_2d = src0.astype(jnp.int32).reshape(ROWS0, CH)
    dst0_2d = dst0.astype(jnp.int32).reshape(ROWS0, CH)
    src1_2d = src1.astype(jnp.int32).reshape(ROWS1, CH)
    dst1_2d = dst1.astype(jnp.int32).reshape(ROWS1, CH)

    zeros0 = jnp.zeros((ACC0 // NS, D), jnp.float32)
    zeros1 = jnp.zeros((ACC1 // NS, D), jnp.float32)

    s0 = bn0_w / jnp.sqrt(1.0 + BN_EPS)
    w0a = W0a.T * s0[None, :]
    c0a = (b0a * s0 + bn0_b).reshape(1, D)
    w0b = W0b.T
    c0b = b0b.reshape(1, D)
    s1 = bn1_w / jnp.sqrt(1.0 + BN_EPS)
    w1a = W1a.T * s1[None, :]
    c1a = (b1a * s1 + bn1_b).reshape(1, D)
    w1b = W1b.T
    c1b = b1b.reshape(1, D)
    l1 = lin1_w.T
    c1 = lin1_b.reshape(1, D)
    l2 = lin2_w.T
    c2 = lin2_b.reshape(1, lin2_w.shape[0])

    agg0 = _agg_mesh_kernel(_agg0_body, N1, ACC0 // NS)
    aggr0 = agg0(x, src0_2d, dst0_2d, zeros0)
    h = _mlp(aggr0, x[:N1], w0a, c0a, w0b, c0b, N1, 2000)

    agg1 = _agg_mesh_kernel(_agg1_body, NC * N2, ACC1 // NS)
    parts = agg1(h, src1_2d, dst1_2d, zeros1)
    out = _head(parts[:N2], parts[N2:], h[:N2],
                w1a, c1a, w1b, c1b, l1, c1, l2, c2, N2, 1024, lin2_w.shape[0])
    return out


# speedup vs baseline: 6.8362x; 1.7691x over previous
import jax
import jax.numpy as jnp
from jax import lax
from jax.experimental import pallas as pl
from jax.experimental.pallas import tpu as pltpu
from jax.experimental.pallas import tpu_sc as plsc

N0, N1, N2 = 100000, 20000, 4096
D = 128
E0, E1 = 320000, 65536
BN_EPS = 1e-5

NC, NS, L = 2, 16, 16
CH = 128

HALF0 = N1 // NC
DUMMY0 = HALF0
ACC0 = 10240
ROWS0 = E0 // CH
ITER0 = (ROWS0 + NS - 1) // NS

ACC1 = N2
ROWS1 = E1 // CH
ITER1 = ROWS1 // (NC * NS)


def _make_agg_body(rowfn, validfn, remapfn, flushfn, iters):
    def body(x_hbm, src_hbm, dst_hbm, zeros_hbm, out_hbm,
             sidxA, sidxB, drawA, drawB, didxA, didxB, rowsA, rowsB,
             acc, semiA, semiB, semgA, semgB):
        cid = lax.axis_index("c")
        sid = lax.axis_index("s")
        sidx = (sidxA, sidxB)
        draw = (drawA, drawB)
        didx = (didxA, didxB)
        rows = (rowsA, rowsB)
        semi = (semiA, semiB)
        semg = (semgA, semgB)

        def start_idx(j, p):
            @pl.when(validfn(cid, sid, j))
            def _():
                r = rowfn(cid, sid, j)
                pltpu.async_copy(src_hbm.at[r], sidx[p], semi[p])
                pltpu.async_copy(dst_hbm.at[r], draw[p], semi[p])

        def wait_idx(j, p):
            @pl.when(validfn(cid, sid, j))
            def _():
                r = rowfn(cid, sid, j)
                pltpu.make_async_copy(src_hbm.at[r], sidx[p], semi[p]).wait()
                pltpu.make_async_copy(dst_hbm.at[r], draw[p], semi[p]).wait()

        def start_gather(j, p):
            @pl.when(validfn(cid, sid, j))
            def _():
                pltpu.async_copy(x_hbm.at[sidx[p]], rows[p], semg[p])

        def step(j, p):
            valid = validfn(cid, sid, j)

            @pl.when(valid)
            def _():
                pltpu.make_async_copy(x_hbm.at[sidx[p]], rows[p], semg[p]).wait()
                for k in range(CH // L):
                    d = draw[p][pl.ds(k * L, L)]
                    didx[p][0, pl.ds(k * L, L)] = remapfn(cid, d)

            start_idx(j + 2, p)
            wait_idx(j + 1, 1 - p)
            start_gather(j + 1, 1 - p)

            @pl.when(valid)
            def _():
                pltpu.sync_copy(rows[p], acc.at[didx[p].at[0]], add=True)

        start_idx(0, 0)
        start_idx(1, 1)
        zrows = acc.shape[0] // NS
        pltpu.sync_copy(zeros_hbm, acc.at[pl.ds(sid * zrows, zrows)])
        wait_idx(0, 0)
        start_gather(0, 0)
        plsc.subcore_barrier()

        def loop(ii, carry):
            step(2 * ii, 0)
            step(2 * ii + 1, 1)
            return carry

        lax.fori_loop(0, (iters + 1) // 2, loop, 0)
        plsc.subcore_barrier()
        flushfn(cid, sid, acc, out_hbm)

    return body


def _agg_kernel(body, out_rows, acc_rows):
    mesh = plsc.VectorSubcoreMesh(core_axis_name="c", subcore_axis_name="s")
    return pl.kernel(
        body,
        out_type=jax.ShapeDtypeStruct((out_rows, D), jnp.float32),
        mesh=mesh,
        scratch_types=[
            pltpu.VMEM((CH,), jnp.int32), pltpu.VMEM((CH,), jnp.int32),
            pltpu.VMEM((CH,), jnp.int32), pltpu.VMEM((CH,), jnp.int32),
            pltpu.VMEM((1, CH), jnp.int32), pltpu.VMEM((1, CH), jnp.int32),
            pltpu.VMEM((CH, D), jnp.float32), pltpu.VMEM((CH, D), jnp.float32),
            pltpu.VMEM_SHARED((acc_rows, D), jnp.float32),
            pltpu.SemaphoreType.DMA, pltpu.SemaphoreType.DMA,
            pltpu.SemaphoreType.DMA, pltpu.SemaphoreType.DMA,
        ],
    )


def _remap0(cid, d):
    local = d - cid * HALF0
    oob = (local < 0) | (local >= HALF0)
    return jnp.where(oob, DUMMY0, local)


def _flush0(cid, sid, acc, out_hbm):
    @pl.when(sid < NS - 1)
    def _():
        pltpu.sync_copy(acc.at[pl.ds(sid * 632, 632)],
                        out_hbm.at[pl.ds(cid * HALF0 + sid * 632, 632)])

    @pl.when(sid == NS - 1)
    def _():
        pltpu.sync_copy(acc.at[pl.ds((NS - 1) * 632, 520)],
                        out_hbm.at[pl.ds(cid * HALF0 + (NS - 1) * 632, 520)])


def _flush1(cid, sid, acc, out_hbm):
    frows = ACC1 // NS
    pltpu.sync_copy(acc.at[pl.ds(sid * frows, frows)],
                    out_hbm.at[pl.ds(cid * ACC1 + sid * frows, frows)])


_agg0_body = _make_agg_body(
    rowfn=lambda cid, sid, j: sid + NS * j,
    validfn=lambda cid, sid, j: sid + NS * j < ROWS0,
    remapfn=_remap0,
    flushfn=_flush0,
    iters=ITER0,
)

_agg1_body = _make_agg_body(
    rowfn=lambda cid, sid, j: cid * (ROWS1 // NC) + sid + NS * j,
    validfn=lambda cid, sid, j: j < ITER1,
    remapfn=lambda cid, d: d,
    flushfn=_flush1,
    iters=ITER1,
)


def _mlp_body(a_ref, x_ref, w1_ref, b1_ref, w2_ref, b2_ref, o_ref):
    h = a_ref[...] + x_ref[...]
    h = jnp.dot(h, w1_ref[...], preferred_element_type=jnp.float32) + b1_ref[...]
    h = jnp.maximum(h, 0.0)
    h = jnp.dot(h, w2_ref[...], preferred_element_type=jnp.float32) + b2_ref[...]
    o_ref[...] = jnp.maximum(h, 0.0)


def _head_body(p0_ref, p1_ref, x_ref, w1_ref, b1_ref, w2_ref, b2_ref,
               l1_ref, c1_ref, l2_ref, c2_ref, o_ref):
    h = p0_ref[...] + p1_ref[...] + x_ref[...]
    h = jnp.dot(h, w1_ref[...], preferred_element_type=jnp.float32) + b1_ref[...]
    h = jnp.maximum(h, 0.0)
    h = jnp.dot(h, w2_ref[...], preferred_element_type=jnp.float32) + b2_ref[...]
    h = jnp.maximum(h, 0.0)
    h = jnp.dot(h, l1_ref[...], preferred_element_type=jnp.float32) + c1_ref[...]
    h = jnp.maximum(h, 0.0)
    z = jnp.dot(h, l2_ref[...], preferred_element_type=jnp.float32) + c2_ref[...]
    m = jnp.max(z, axis=-1, keepdims=True)
    e = jnp.exp(z - m)
    s = jnp.sum(e, axis=-1, keepdims=True)
    o_ref[...] = z - m - jnp.log(s)


def _full(shape):
    return pl.BlockSpec(shape, lambda i: (0, 0))


def _row(block, width=D, off=0):
    return pl.BlockSpec((block, width), lambda i, o=off: (i + o, 0))


def _mlp(aggr, x_full, w1, b1, w2, b2, rows, block):
    return pl.pallas_call(
        _mlp_body,
        grid=(rows // block,),
        in_specs=[_row(block), _row(block), _full((D, D)), _full((1, D)),
                  _full((D, D)), _full((1, D))],
        out_specs=_row(block),
        out_shape=jax.ShapeDtypeStruct((rows, D), jnp.float32),
    )(aggr, x_full, w1, b1, w2, b2)


def _head(parts, h_full, w1, b1, w2, b2, l1, c1, l2, c2, rows, block, dout):
    return pl.pallas_call(
        _head_body,
        grid=(rows // block,),
        in_specs=[_row(block), _row(block, off=N2 // block), _row(block),
                  _full((D, D)), _full((1, D)), _full((D, D)), _full((1, D)),
                  _full((D, D)), _full((1, D)), _full((D, dout)), _full((1, dout))],
        out_specs=pl.BlockSpec((block, dout), lambda i: (i, 0)),
        out_shape=jax.ShapeDtypeStruct((rows, dout), jnp.float32),
    )(parts, parts, h_full, w1, b1, w2, b2, l1, c1, l2, c2)


@jax.jit
def kernel(x, src0, dst0, src1, dst1, W0a, b0a, bn0_w, bn0_b, W0b, b0b,
           W1a, b1a, bn1_w, bn1_b, W1b, b1b, lin1_w, lin1_b, lin2_w, lin2_b):
    x = x.astype(jnp.float32)
    src0_2d = src0.astype(jnp.int32).reshape(ROWS0, CH)
    dst0_2d = dst0.astype(jnp.int32).reshape(ROWS0, CH)
    src1_2d = src1.astype(jnp.int32).reshape(ROWS1, CH)
    dst1_2d = dst1.astype(jnp.int32).reshape(ROWS1, CH)

    zeros0 = jnp.zeros((ACC0 // NS, D), jnp.float32)
    zeros1 = jnp.zeros((ACC1 // NS, D), jnp.float32)

    s0 = bn0_w / jnp.sqrt(1.0 + BN_EPS)
    w0a = W0a.T * s0[None, :]
    c0a = (b0a * s0 + bn0_b).reshape(1, D)
    s1 = bn1_w / jnp.sqrt(1.0 + BN_EPS)
    w1a = W1a.T * s1[None, :]
    c1a = (b1a * s1 + bn1_b).reshape(1, D)
    dout = lin2_w.shape[0]

    agg0 = _agg_kernel(_agg0_body, N1, ACC0)
    aggr0 = agg0(x, src0_2d, dst0_2d, zeros0)
    h = _mlp(aggr0, x, w0a, c0a, W0b.T, b0b.reshape(1, D), N1, 2000)

    agg1 = _agg_kernel(_agg1_body, NC * N2, ACC1)
    parts = agg1(h, src1_2d, dst1_2d, zeros1)
    out = _head(parts, h, w1a, c1a, W1b.T, b1b.reshape(1, D),
                lin1_w.T, lin1_b.reshape(1, D), lin2_w.T,
                lin2_b.reshape(1, dout), N2, 1024, dout)
    return out
